# Initial kernel scaffold; baseline (speedup 1.0000x reference)
import jax
import jax.numpy as jnp
from jax.experimental import pallas as pl


def kernel(inputs, codebook):
    raise NotImplementedError("write your pallas kernel here")


# speedup vs baseline: 1.4851x; 1.4851x over previous
import functools

import jax
import jax.numpy as jnp
from jax import lax
from jax.experimental import pallas as pl
from jax.experimental.pallas import tpu as pltpu
from jax.experimental.pallas import tpu_sc as plsc

B = 16
D = 256
L = 1024
K = 8192
N_TOK = B * L
WINDOWS = ((0, 2736), (2736, 2736), (5472, 2720))
LOSS_SCALE = (1.0 + 0.25) / float(N_TOK * D)


def _dist_argmin_body(x_ref, cb_ref, xn_ref, cn_ref, idx_ref, loss_ref):
    b = pl.program_id(0)
    x = x_ref[0]
    xbf = x.astype(jnp.bfloat16)
    xn = xn_ref[0, 0, :]

    acc_vb = jnp.full((L,), jnp.inf, jnp.bfloat16)
    acc_vf = jnp.full((L,), jnp.inf, jnp.float32)
    acc_i = jnp.zeros((L,), jnp.int32)

    for start, size in WINDOWS:
        c = cb_ref[pl.ds(start, size), :]
        cn = cn_ref[0, pl.ds(start, size)]
        mm = lax.dot_general(
            c.astype(jnp.bfloat16), xbf, (((1,), (0,)), ((), ())),
            preferred_element_type=jnp.float32)
        d = (xn[None, :] + cn[:, None]) - 2.0 * mm
        vmin = jnp.min(d, axis=0)
        rowi = lax.broadcasted_iota(jnp.int32, d.shape, 0) + start
        imin = jnp.min(jnp.where(d == vmin[None, :], rowi, K), axis=0)
        accf = acc_vb.astype(jnp.float32)
        keep = (accf < vmin) | jnp.isnan(accf) | ((accf == vmin) & (acc_i < imin))
        acc_i = jnp.where(keep, acc_i, imin)
        acc_vf = jnp.where(keep, acc_vf, vmin)
        acc_vb = jnp.where(keep, acc_vb, vmin.astype(jnp.bfloat16))

    idx_ref[0, 0, :] = acc_i

    @pl.when(b == 0)
    def _init():
        loss_ref[0, 0] = jnp.float32(0.0)

    loss_ref[0, 0] = loss_ref[0, 0] + jnp.sum(acc_vf)

    @pl.when(b == pl.num_programs(0) - 1)
    def _finish():
        loss_ref[0, 0] = loss_ref[0, 0] * jnp.float32(LOSS_SCALE)


def _transpose_body(q_ref, x_ref, out_ref):
    qt = jnp.transpose(q_ref[:], (1, 0))
    x = x_ref[0]
    out_ref[0] = x + (qt - x)


_NC, _NS = 2, 16
_NW = _NC * _NS
_PER_W = N_TOK // _NW
_GCH = 128
_NCH = _PER_W // _GCH


def _sc_gather_body(table_hbm, idx_hbm, out_hbm, idx_v, rows_v, sem):
    wid = lax.axis_index("s") * _NC + lax.axis_index("c")
    base = wid * _PER_W

    def body(ch, carry):
        off = base + ch * _GCH
        pltpu.sync_copy(idx_hbm.at[pl.ds(off, _GCH)], idx_v)
        pltpu.async_copy(table_hbm.at[idx_v], rows_v, sem).wait()
        pltpu.sync_copy(rows_v, out_hbm.at[pl.ds(off, _GCH)])
        return carry

    lax.fori_loop(0, _NCH, body, 0)


def _make_sc_gather():
    return pl.kernel(
        _sc_gather_body,
        out_type=jax.ShapeDtypeStruct((N_TOK, D), jnp.float32),
        mesh=plsc.VectorSubcoreMesh(core_axis_name="c", subcore_axis_name="s",
                                    num_cores=_NC, num_subcores=_NS),
        scratch_types=[
            pltpu.VMEM((_GCH,), jnp.int32),
            pltpu.VMEM((_GCH, D), jnp.float32),
            pltpu.SemaphoreType.DMA,
        ],
    )


def kernel(inputs, codebook):
    inputs_permuted = jnp.transpose(inputs, (0, 2, 1))
    xnorm = jnp.sum(inputs_permuted ** 2, axis=2)
    cnorm = jnp.sum(codebook ** 2, axis=1)

    idx3, loss_acc = pl.pallas_call(
        _dist_argmin_body,
        grid=(B,),
        in_specs=[
            pl.BlockSpec((1, D, L), lambda i: (i, 0, 0)),
            pl.BlockSpec((K, D), lambda i: (0, 0)),
            pl.BlockSpec((1, 1, L), lambda i: (i, 0, 0)),
            pl.BlockSpec((1, K), lambda i: (0, 0)),
        ],
        out_specs=[
            pl.BlockSpec((1, 1, L), lambda i: (i, 0, 0)),
            pl.BlockSpec(memory_space=pltpu.SMEM),
        ],
        out_shape=[
            jax.ShapeDtypeStruct((B, 1, L), jnp.int32),
            jax.ShapeDtypeStruct((1, 1), jnp.float32),
        ],
    )(inputs, codebook, xnorm.reshape(B, 1, L), cnorm.reshape(1, K))

    indices = idx3.reshape(B, L)
    q_flat = _make_sc_gather()(codebook, idx3.reshape(N_TOK))

    quantized = pl.pallas_call(
        _transpose_body,
        grid=(B,),
        in_specs=[pl.BlockSpec((L, D), lambda i: (i, 0)),
                  pl.BlockSpec((1, D, L), lambda i: (i, 0, 0))],
        out_specs=pl.BlockSpec((1, D, L), lambda i: (i, 0, 0)),
        out_shape=jax.ShapeDtypeStruct((B, D, L), jnp.float32),
    )(q_flat, inputs)

    return quantized, loss_acc[0, 0], indices


# speedup vs baseline: 1.5042x; 1.0128x over previous
import functools

import jax
import jax.numpy as jnp
from jax import lax
from jax.experimental import pallas as pl
from jax.experimental.pallas import tpu as pltpu
from jax.experimental.pallas import tpu_sc as plsc

B = 16
D = 256
L = 1024
K = 8192
N_TOK = B * L
WINDOWS = ((0, 2736), (2736, 2736), (5472, 2720))
LOSS_SCALE = (1.0 + 0.25) / float(N_TOK * D)


def _dist_argmin_body(x_ref, cb_ref, xn_ref, cn_ref, idx_ref, loss_ref):
    b = pl.program_id(0)
    xbf = x_ref[0].astype(jnp.bfloat16)
    xn = xn_ref[0, 0, :]

    acc_vb = jnp.full((L,), jnp.inf, jnp.bfloat16)
    acc_vf = jnp.full((L,), jnp.inf, jnp.float32)
    acc_i = jnp.zeros((L,), jnp.int32)

    for start, size in WINDOWS:
        c = cb_ref[pl.ds(start, size), :]
        cn = cn_ref[0, pl.ds(start, size)]
        mm = lax.dot_general(
            c, xbf, (((1,), (0,)), ((), ())),
            preferred_element_type=jnp.float32)
        d = (xn[None, :] + cn[:, None]) - 2.0 * mm
        vmin = jnp.min(d, axis=0)
        rowi = lax.broadcasted_iota(jnp.int32, d.shape, 0) + start
        imin = jnp.min(jnp.where(d == vmin[None, :], rowi, K), axis=0)
        accf = acc_vb.astype(jnp.float32)
        keep = (accf < vmin) | jnp.isnan(accf) | ((accf == vmin) & (acc_i < imin))
        acc_i = jnp.where(keep, acc_i, imin)
        acc_vf = jnp.where(keep, acc_vf, vmin)
        acc_vb = jnp.where(keep, acc_vb, vmin.astype(jnp.bfloat16))

    idx_ref[0, 0, :] = acc_i

    @pl.when(b == 0)
    def _init():
        loss_ref[0, 0] = jnp.float32(0.0)

    loss_ref[0, 0] = loss_ref[0, 0] + jnp.sum(acc_vf)

    @pl.when(b == pl.num_programs(0) - 1)
    def _finish():
        loss_ref[0, 0] = loss_ref[0, 0] * jnp.float32(LOSS_SCALE)


def _transpose_body(q_ref, x_ref, out_ref):
    qt = jnp.transpose(q_ref[:], (1, 0))
    x = x_ref[0]
    out_ref[0] = x + (qt - x)


_NC, _NS = 2, 16
_NW = _NC * _NS
_PER_W = N_TOK // _NW
_GCH = 128
_NCH = _PER_W // _GCH


def _sc_gather_body(table_hbm, idx_hbm, out_hbm, idx_v, rows_v, sem):
    wid = lax.axis_index("s") * _NC + lax.axis_index("c")
    base = wid * _PER_W

    def body(ch, carry):
        off = base + ch * _GCH
        pltpu.sync_copy(idx_hbm.at[pl.ds(off, _GCH)], idx_v)
        pltpu.async_copy(table_hbm.at[idx_v], rows_v, sem).wait()
        pltpu.sync_copy(rows_v, out_hbm.at[pl.ds(off, _GCH)])
        return carry

    lax.fori_loop(0, _NCH, body, 0)


def _make_sc_gather():
    return pl.kernel(
        _sc_gather_body,
        out_type=jax.ShapeDtypeStruct((N_TOK, D), jnp.float32),
        mesh=plsc.VectorSubcoreMesh(core_axis_name="c", subcore_axis_name="s",
                                    num_cores=_NC, num_subcores=_NS),
        scratch_types=[
            pltpu.VMEM((_GCH,), jnp.int32),
            pltpu.VMEM((_GCH, D), jnp.float32),
            pltpu.SemaphoreType.DMA,
        ],
    )


def kernel(inputs, codebook):
    inputs_permuted = jnp.transpose(inputs, (0, 2, 1))
    xnorm = jnp.sum(inputs_permuted ** 2, axis=2)
    cnorm = jnp.sum(codebook ** 2, axis=1)
    codebook_bf = codebook.astype(jnp.bfloat16)

    idx3, loss_acc = pl.pallas_call(
        _dist_argmin_body,
        grid=(B,),
        in_specs=[
            pl.BlockSpec((1, D, L), lambda i: (i, 0, 0)),
            pl.BlockSpec((K, D), lambda i: (0, 0)),
            pl.BlockSpec((1, 1, L), lambda i: (i, 0, 0)),
            pl.BlockSpec((1, K), lambda i: (0, 0)),
        ],
        out_specs=[
            pl.BlockSpec((1, 1, L), lambda i: (i, 0, 0)),
            pl.BlockSpec(memory_space=pltpu.SMEM),
        ],
        out_shape=[
            jax.ShapeDtypeStruct((B, 1, L), jnp.int32),
            jax.ShapeDtypeStruct((1, 1), jnp.float32),
        ],
    )(inputs, codebook_bf, xnorm.reshape(B, 1, L), cnorm.reshape(1, K))

    indices = idx3.reshape(B, L)
    q_flat = _make_sc_gather()(codebook, idx3.reshape(N_TOK))

    quantized = pl.pallas_call(
        _transpose_body,
        grid=(B,),
        in_specs=[pl.BlockSpec((L, D), lambda i: (i, 0)),
                  pl.BlockSpec((1, D, L), lambda i: (i, 0, 0))],
        out_specs=pl.BlockSpec((1, D, L), lambda i: (i, 0, 0)),
        out_shape=jax.ShapeDtypeStruct((B, D, L), jnp.float32),
    )(q_flat, inputs)

    return quantized, loss_acc[0, 0], indices


# speedup vs baseline: 1.5718x; 1.0449x over previous
import functools

import jax
import jax.numpy as jnp
from jax import lax
from jax.experimental import pallas as pl
from jax.experimental.pallas import tpu as pltpu
from jax.experimental.pallas import tpu_sc as plsc

B = 16
D = 256
L = 1024
K = 8192
N_TOK = B * L
WINDOWS = ((0, 2736), (2736, 2736), (5472, 2720))
LOSS_SCALE = (1.0 + 0.25) / float(N_TOK * D)


def _dist_argmin_body(x_ref, cb_ref, xn_ref, cn_ref, idx_ref, loss_ref):
    b = pl.program_id(0)
    xbf = x_ref[0].astype(jnp.bfloat16)
    xn = xn_ref[0, 0, :]

    acc_vb = jnp.full((L,), jnp.inf, jnp.bfloat16)
    acc_vf = jnp.full((L,), jnp.inf, jnp.float32)
    acc_i = jnp.zeros((L,), jnp.int32)
    rowi_full = lax.broadcasted_iota(jnp.int32, (WINDOWS[0][1], L), 0)

    for start, size in WINDOWS:
        c = cb_ref[pl.ds(start, size), :]
        cn = cn_ref[0, pl.ds(start, size)]
        mm2 = lax.dot_general(
            c, xbf, (((1,), (0,)), ((), ())),
            preferred_element_type=jnp.float32)
        d = (xn[None, :] + cn[:, None]) - mm2
        vmin = jnp.min(d, axis=0)
        rowi = rowi_full[:size] if size < WINDOWS[0][1] else rowi_full
        imin = jnp.min(jnp.where(d == vmin[None, :], rowi, K), axis=0) + start
        accf = acc_vb.astype(jnp.float32)
        keep = (accf < vmin) | jnp.isnan(accf) | ((accf == vmin) & (acc_i < imin))
        acc_i = jnp.where(keep, acc_i, imin)
        acc_vf = jnp.where(keep, acc_vf, vmin)
        acc_vb = jnp.where(keep, acc_vb, vmin.astype(jnp.bfloat16))

    idx_ref[0, 0, :] = acc_i

    @pl.when(b == 0)
    def _init():
        loss_ref[0, 0] = jnp.float32(0.0)

    loss_ref[0, 0] = loss_ref[0, 0] + jnp.sum(acc_vf)

    @pl.when(b == pl.num_programs(0) - 1)
    def _finish():
        loss_ref[0, 0] = loss_ref[0, 0] * jnp.float32(LOSS_SCALE)


def _transpose_body(q_ref, x_ref, out_ref):
    qt = jnp.transpose(q_ref[:], (1, 0))
    x = x_ref[0]
    out_ref[0] = x + (qt - x)


_NC, _NS = 2, 16
_NW = _NC * _NS
_PER_W = N_TOK // _NW
_GCH = 128
_NCH = _PER_W // _GCH


def _sc_gather_body(table_hbm, idx_hbm, out_hbm, idx_v, rows_v, sem):
    wid = lax.axis_index("s") * _NC + lax.axis_index("c")
    base = wid * _PER_W

    def body(ch, carry):
        off = base + ch * _GCH
        pltpu.sync_copy(idx_hbm.at[pl.ds(off, _GCH)], idx_v)
        pltpu.async_copy(table_hbm.at[idx_v], rows_v, sem).wait()
        pltpu.sync_copy(rows_v, out_hbm.at[pl.ds(off, _GCH)])
        return carry

    lax.fori_loop(0, _NCH, body, 0)


def _make_sc_gather():
    return pl.kernel(
        _sc_gather_body,
        out_type=jax.ShapeDtypeStruct((N_TOK, D), jnp.float32),
        mesh=plsc.VectorSubcoreMesh(core_axis_name="c", subcore_axis_name="s",
                                    num_cores=_NC, num_subcores=_NS),
        scratch_types=[
            pltpu.VMEM((_GCH,), jnp.int32),
            pltpu.VMEM((_GCH, D), jnp.float32),
            pltpu.SemaphoreType.DMA,
        ],
    )


def kernel(inputs, codebook):
    inputs_permuted = jnp.transpose(inputs, (0, 2, 1))
    xnorm = jnp.sum(inputs_permuted ** 2, axis=2)
    cnorm = jnp.sum(codebook ** 2, axis=1)
    codebook_bf = codebook.astype(jnp.bfloat16) * jnp.bfloat16(2.0)

    idx3, loss_acc = pl.pallas_call(
        _dist_argmin_body,
        grid=(B,),
        in_specs=[
            pl.BlockSpec((1, D, L), lambda i: (i, 0, 0)),
            pl.BlockSpec((K, D), lambda i: (0, 0)),
            pl.BlockSpec((1, 1, L), lambda i: (i, 0, 0)),
            pl.BlockSpec((1, K), lambda i: (0, 0)),
        ],
        out_specs=[
            pl.BlockSpec((1, 1, L), lambda i: (i, 0, 0)),
            pl.BlockSpec(memory_space=pltpu.SMEM),
        ],
        out_shape=[
            jax.ShapeDtypeStruct((B, 1, L), jnp.int32),
            jax.ShapeDtypeStruct((1, 1), jnp.float32),
        ],
    )(inputs, codebook_bf, xnorm.reshape(B, 1, L), cnorm.reshape(1, K))

    indices = idx3.reshape(B, L)
    q_flat = _make_sc_gather()(codebook, idx3.reshape(N_TOK))

    quantized = pl.pallas_call(
        _transpose_body,
        grid=(B,),
        in_specs=[pl.BlockSpec((L, D), lambda i: (i, 0)),
                  pl.BlockSpec((1, D, L), lambda i: (i, 0, 0))],
        out_specs=pl.BlockSpec((1, D, L), lambda i: (i, 0, 0)),
        out_shape=jax.ShapeDtypeStruct((B, D, L), jnp.float32),
    )(q_flat, inputs)

    return quantized, loss_acc[0, 0], indices


# speedup vs baseline: 1.6557x; 1.0534x over previous
import functools

import jax
import jax.numpy as jnp
from jax import lax
from jax.experimental import pallas as pl
from jax.experimental.pallas import tpu as pltpu
from jax.experimental.pallas import tpu_sc as plsc

B = 16
D = 256
L = 1024
K = 8192
N_TOK = B * L
WINDOWS = ((0, 2736), (2736, 2736), (5472, 2720))
LOSS_SCALE = (1.0 + 0.25) / float(N_TOK * D)


def _dist_argmin_body(x_ref, cb_ref, xn_ref, cn_ref, idx_ref, loss_ref):
    b = pl.program_id(0)
    xbf = x_ref[0].astype(jnp.bfloat16)
    xn = xn_ref[0, 0, :]

    acc_vb = jnp.full((L,), jnp.inf, jnp.bfloat16)
    acc_vf = jnp.full((L,), jnp.inf, jnp.float32)
    acc_i = jnp.zeros((L,), jnp.int32)
    rowi_full = lax.broadcasted_iota(
        jnp.int32, (WINDOWS[0][1], L), 0).astype(jnp.float32)

    for start, size in WINDOWS:
        c = cb_ref[pl.ds(start, size), :]
        cn = cn_ref[0, pl.ds(start, size)]
        mm2 = lax.dot_general(
            c, xbf, (((1,), (0,)), ((), ())),
            preferred_element_type=jnp.float32)
        d = (xn[None, :] + cn[:, None]) - mm2
        vmin = jnp.min(d, axis=0)
        rowi = rowi_full[:size] if size < WINDOWS[0][1] else rowi_full
        imin_f = jnp.min(jnp.where(d == vmin[None, :], rowi, jnp.float32(K)),
                         axis=0)
        imin = imin_f.astype(jnp.int32) + start
        accf = acc_vb.astype(jnp.float32)
        keep = (accf < vmin) | jnp.isnan(accf) | ((accf == vmin) & (acc_i < imin))
        acc_i = jnp.where(keep, acc_i, imin)
        acc_vf = jnp.where(keep, acc_vf, vmin)
        acc_vb = jnp.where(keep, acc_vb, vmin.astype(jnp.bfloat16))

    idx_ref[0, 0, :] = acc_i

    @pl.when(b == 0)
    def _init():
        loss_ref[0, 0] = jnp.float32(0.0)

    loss_ref[0, 0] = loss_ref[0, 0] + jnp.sum(acc_vf)


def _transpose_body(q_ref, x_ref, out_ref):
    qt = jnp.transpose(q_ref[:], (1, 0))
    x = x_ref[0]
    out_ref[0] = x + (qt - x)


_BH = B // 2
_NTOK_H = _BH * L
_NC, _NS = 2, 16
_NW = _NC * _NS
_PER_W = _NTOK_H // _NW
_GCH = 128
_NCH = _PER_W // _GCH


def _sc_gather_body(table_hbm, idx_hbm, out_hbm, idx_v, rows_v, sem):
    wid = lax.axis_index("s") * _NC + lax.axis_index("c")
    base = wid * _PER_W

    def body(ch, carry):
        off = base + ch * _GCH
        pltpu.sync_copy(idx_hbm.at[pl.ds(off, _GCH)], idx_v)
        pltpu.async_copy(table_hbm.at[idx_v], rows_v, sem).wait()
        pltpu.sync_copy(rows_v, out_hbm.at[pl.ds(off, _GCH)])
        return carry

    lax.fori_loop(0, _NCH, body, 0)


def _make_sc_gather():
    return pl.kernel(
        _sc_gather_body,
        out_type=jax.ShapeDtypeStruct((_NTOK_H, D), jnp.float32),
        mesh=plsc.VectorSubcoreMesh(core_axis_name="c", subcore_axis_name="s",
                                    num_cores=_NC, num_subcores=_NS),
        scratch_types=[
            pltpu.VMEM((_GCH,), jnp.int32),
            pltpu.VMEM((_GCH, D), jnp.float32),
            pltpu.SemaphoreType.DMA,
        ],
    )


def kernel(inputs, codebook):
    inputs_permuted = jnp.transpose(inputs, (0, 2, 1))
    xnorm = jnp.sum(inputs_permuted ** 2, axis=2)
    cnorm = jnp.sum(codebook ** 2, axis=1)
    codebook_bf = codebook.astype(jnp.bfloat16) * jnp.bfloat16(2.0)

    xnorm3 = xnorm.reshape(B, 1, L)
    cnorm2 = cnorm.reshape(1, K)

    def dist_argmin_half(off):
        return pl.pallas_call(
            _dist_argmin_body,
            grid=(_BH,),
            in_specs=[
                pl.BlockSpec((1, D, L), lambda i: (i + off, 0, 0)),
                pl.BlockSpec((K, D), lambda i: (0, 0)),
                pl.BlockSpec((1, 1, L), lambda i: (i + off, 0, 0)),
                pl.BlockSpec((1, K), lambda i: (0, 0)),
            ],
            out_specs=[
                pl.BlockSpec((1, 1, L), lambda i: (i, 0, 0)),
                pl.BlockSpec(memory_space=pltpu.SMEM),
            ],
            out_shape=[
                jax.ShapeDtypeStruct((_BH, 1, L), jnp.int32),
                jax.ShapeDtypeStruct((1, 1), jnp.float32),
            ],
        )(inputs, codebook_bf, xnorm3, cnorm2)

    idx_a, loss_a = dist_argmin_half(0)
    idx_b, loss_b = dist_argmin_half(_BH)

    gather = _make_sc_gather()
    q_a = gather(codebook, idx_a.reshape(_NTOK_H))
    q_b = gather(codebook, idx_b.reshape(_NTOK_H))

    quantized_half = pl.pallas_call(
        _transpose_body,
        grid=(_BH,),
        in_specs=[pl.BlockSpec((L, D), lambda i: (i, 0)),
                  pl.BlockSpec((1, D, L), lambda i: (i, 0, 0))],
        out_specs=pl.BlockSpec((1, D, L), lambda i: (i, 0, 0)),
        out_shape=jax.ShapeDtypeStruct((B, D, L), jnp.float32),
    )(q_a, inputs)

    quantized = pl.pallas_call(
        lambda prev_ref, q_ref, x_ref, out_ref: _transpose_body(
            q_ref, x_ref, out_ref),
        grid=(_BH,),
        in_specs=[pl.BlockSpec(memory_space=pltpu.MemorySpace.HBM),
                  pl.BlockSpec((L, D), lambda i: (i, 0)),
                  pl.BlockSpec((1, D, L), lambda i: (i + _BH, 0, 0))],
        out_specs=pl.BlockSpec((1, D, L), lambda i: (i + _BH, 0, 0)),
        out_shape=jax.ShapeDtypeStruct((B, D, L), jnp.float32),
        input_output_aliases={0: 0},
    )(quantized_half, q_b, inputs)

    indices = jnp.concatenate([idx_a, idx_b], axis=0).reshape(B, L)
    loss = (loss_a[0, 0] + loss_b[0, 0]) * jnp.float32(LOSS_SCALE)
    return quantized, loss, indices


# speedup vs baseline: 1.7145x; 1.0355x over previous
import functools

import jax
import jax.numpy as jnp
from jax import lax
from jax.experimental import pallas as pl
from jax.experimental.pallas import tpu as pltpu
from jax.experimental.pallas import tpu_sc as plsc

B = 16
D = 256
L = 1024
K = 8192
N_TOK = B * L
WINDOWS = ((0, 2736), (2736, 2736), (5472, 2720))
LOSS_SCALE = (1.0 + 0.25) / float(N_TOK * D)


def _dist_argmin_body(x_ref, cb_ref, cn_ref, idx_ref, loss_ref):
    b = pl.program_id(0)
    x = x_ref[0]
    xbf = x.astype(jnp.bfloat16)
    xn = jnp.sum(x * x, axis=0)

    acc_vb = jnp.full((L,), jnp.inf, jnp.bfloat16)
    acc_vf = jnp.full((L,), jnp.inf, jnp.float32)
    acc_i = jnp.zeros((L,), jnp.int32)
    rowi_full = lax.broadcasted_iota(
        jnp.int32, (WINDOWS[0][1], L), 0).astype(jnp.float32)

    for start, size in WINDOWS:
        c = cb_ref[pl.ds(start, size), :]
        cn = cn_ref[0, pl.ds(start, size)]
        mm2 = lax.dot_general(
            c, xbf, (((1,), (0,)), ((), ())),
            preferred_element_type=jnp.float32)
        d = (xn[None, :] + cn[:, None]) - mm2
        vmin = jnp.min(d, axis=0)
        rowi = rowi_full[:size] if size < WINDOWS[0][1] else rowi_full
        imin_f = jnp.min(jnp.where(d == vmin[None, :], rowi, jnp.float32(K)),
                         axis=0)
        imin = imin_f.astype(jnp.int32) + start
        accf = acc_vb.astype(jnp.float32)
        keep = (accf < vmin) | jnp.isnan(accf) | ((accf == vmin) & (acc_i < imin))
        acc_i = jnp.where(keep, acc_i, imin)
        acc_vf = jnp.where(keep, acc_vf, vmin)
        acc_vb = jnp.where(keep, acc_vb, vmin.astype(jnp.bfloat16))

    idx_ref[0, 0, :] = acc_i

    @pl.when(b == 0)
    def _init():
        loss_ref[0, 0] = jnp.float32(0.0)

    loss_ref[0, 0] = loss_ref[0, 0] + jnp.sum(acc_vf)


def _transpose_body(q_ref, x_ref, out_ref):
    qt = jnp.transpose(q_ref[:], (1, 0))
    x = x_ref[0]
    out_ref[0] = x + (qt - x)


_BH = B // 2
_NTOK_H = _BH * L
_NC, _NS = 2, 16
_NW = _NC * _NS
_PER_W = _NTOK_H // _NW
_GCH = 128
_NCH = _PER_W // _GCH


def _sc_gather_body(table_hbm, idx_hbm, out_hbm, idx_v, rows_v, sem):
    wid = lax.axis_index("s") * _NC + lax.axis_index("c")
    base = wid * _PER_W

    def body(ch, carry):
        off = base + ch * _GCH
        pltpu.sync_copy(idx_hbm.at[pl.ds(off, _GCH)], idx_v)
        pltpu.async_copy(table_hbm.at[idx_v], rows_v, sem).wait()
        pltpu.sync_copy(rows_v, out_hbm.at[pl.ds(off, _GCH)])
        return carry

    lax.fori_loop(0, _NCH, body, 0)


def _make_sc_gather():
    return pl.kernel(
        _sc_gather_body,
        out_type=jax.ShapeDtypeStruct((_NTOK_H, D), jnp.float32),
        mesh=plsc.VectorSubcoreMesh(core_axis_name="c", subcore_axis_name="s",
                                    num_cores=_NC, num_subcores=_NS),
        scratch_types=[
            pltpu.VMEM((_GCH,), jnp.int32),
            pltpu.VMEM((_GCH, D), jnp.float32),
            pltpu.SemaphoreType.DMA,
        ],
    )


def kernel(inputs, codebook):
    cnorm = jnp.sum(codebook ** 2, axis=1)
    codebook_bf = codebook.astype(jnp.bfloat16) * jnp.bfloat16(2.0)

    cnorm2 = cnorm.reshape(1, K)

    def dist_argmin_half(off):
        return pl.pallas_call(
            _dist_argmin_body,
            grid=(_BH,),
            in_specs=[
                pl.BlockSpec((1, D, L), lambda i: (i + off, 0, 0)),
                pl.BlockSpec((K, D), lambda i: (0, 0)),
                pl.BlockSpec((1, K), lambda i: (0, 0)),
            ],
            out_specs=[
                pl.BlockSpec((1, 1, L), lambda i: (i, 0, 0)),
                pl.BlockSpec(memory_space=pltpu.SMEM),
            ],
            out_shape=[
                jax.ShapeDtypeStruct((_BH, 1, L), jnp.int32),
                jax.ShapeDtypeStruct((1, 1), jnp.float32),
            ],
        )(inputs, codebook_bf, cnorm2)

    idx_a, loss_a = dist_argmin_half(0)
    idx_b, loss_b = dist_argmin_half(_BH)

    gather = _make_sc_gather()
    q_a = gather(codebook, idx_a.reshape(_NTOK_H))
    q_b = gather(codebook, idx_b.reshape(_NTOK_H))

    quantized_half = pl.pallas_call(
        _transpose_body,
        grid=(_BH,),
        in_specs=[pl.BlockSpec((L, D), lambda i: (i, 0)),
                  pl.BlockSpec((1, D, L), lambda i: (i, 0, 0))],
        out_specs=pl.BlockSpec((1, D, L), lambda i: (i, 0, 0)),
        out_shape=jax.ShapeDtypeStruct((B, D, L), jnp.float32),
    )(q_a, inputs)

    quantized = pl.pallas_call(
        lambda prev_ref, q_ref, x_ref, out_ref: _transpose_body(
            q_ref, x_ref, out_ref),
        grid=(_BH,),
        in_specs=[pl.BlockSpec(memory_space=pltpu.MemorySpace.HBM),
                  pl.BlockSpec((L, D), lambda i: (i, 0)),
                  pl.BlockSpec((1, D, L), lambda i: (i + _BH, 0, 0))],
        out_specs=pl.BlockSpec((1, D, L), lambda i: (i + _BH, 0, 0)),
        out_shape=jax.ShapeDtypeStruct((B, D, L), jnp.float32),
        input_output_aliases={0: 0},
    )(quantized_half, q_b, inputs)

    indices = jnp.concatenate([idx_a, idx_b], axis=0).reshape(B, L)
    loss = (loss_a[0, 0] + loss_b[0, 0]) * jnp.float32(LOSS_SCALE)
    return quantized, loss, indices
